# baseline (speedup 1.0000x reference)
import functools

import jax
import jax.numpy as jnp
from jax import lax
from jax.experimental import pallas as pl
from jax.experimental.pallas import tpu as pltpu
from jax.experimental.pallas import tpu_sc as plsc

N = 10000
E = 320000
D_IN = 128
D_H = 128
D_OUT = 40
D_OUT_PAD = 48

NC = 2
NS = 16
NW = NC * NS
ROWS_PER_TILE = 640
N_PAD = NS * ROWS_PER_TILE
EDGES_PER_W = E // NW
CHUNK = 80
NCHUNK = EDGES_PER_W // CHUNK

R_BLK = 2048
GRID_R = 5


def _degree_body(edges_hbm, zeros_hbm, hs_out, hd_out, hist_s, hist_d,
                 sidx2, didx2):
    c = lax.axis_index("c")
    s = lax.axis_index("s")
    wid = c * NS + s
    pltpu.sync_copy(zeros_hbm, hist_s)
    pltpu.sync_copy(zeros_hbm, hist_d)
    pltpu.sync_copy(edges_hbm.at[0, wid], sidx2)
    pltpu.sync_copy(edges_hbm.at[1, wid], didx2)
    ones = jnp.ones((16,), jnp.float32)

    def body(i, carry):
        for j in range(CHUNK // 16):
            si = sidx2[i, pl.ds(j * 16, 16)]
            di = didx2[i, pl.ds(j * 16, 16)]
            plsc.addupdate_scatter(hist_s, [si], ones)
            plsc.addupdate_scatter(hist_d, [di], ones)
        return carry

    lax.fori_loop(0, NCHUNK, body, 0)
    pltpu.sync_copy(hist_s, hs_out.at[wid])
    pltpu.sync_copy(hist_d, hd_out.at[wid])


def _degrees(edges3, zeros1d):
    k = pl.kernel(
        _degree_body,
        out_type=(
            jax.ShapeDtypeStruct((NW, N_PAD), jnp.float32),
            jax.ShapeDtypeStruct((NW, N_PAD), jnp.float32),
        ),
        mesh=plsc.VectorSubcoreMesh(core_axis_name="c", subcore_axis_name="s"),
        scratch_types=[
            pltpu.VMEM((N_PAD,), jnp.float32),
            pltpu.VMEM((N_PAD,), jnp.float32),
            pltpu.VMEM((NCHUNK, CHUNK), jnp.int32),
            pltpu.VMEM((NCHUNK, CHUNK), jnp.int32),
        ],
        compiler_params=pltpu.CompilerParams(needs_layout_passes=False),
    )
    return k(edges3, zeros1d)


def _prop_body(nbuf, xs_hbm, edges_hbm, zeros_hbm, parts_out,
               acc, sidxb, didxb, rows, isems, isemd, gsem, ssem):
    c = lax.axis_index("c")
    s = lax.axis_index("s")
    wid = c * NS + s
    def fetch_idx(j, b):
        pltpu.async_copy(edges_hbm.at[0, wid, j], sidxb.at[b], isems.at[b])
        pltpu.async_copy(edges_hbm.at[1, wid, j], didxb.at[b], isemd.at[b])

    def wait_idx(b):
        pltpu.make_async_copy(edges_hbm.at[0, wid, 0], sidxb.at[b],
                              isems.at[b]).wait()
        pltpu.make_async_copy(edges_hbm.at[1, wid, 0], didxb.at[b],
                              isemd.at[b]).wait()

    fetch_idx(0, 0)
    fetch_idx(1, 1)
    wait_idx(0)
    pltpu.async_copy(xs_hbm.at[sidxb.at[0]], rows.at[0], gsem.at[0])
    pltpu.sync_copy(zeros_hbm.at[0],
                    acc.at[pl.ds(s * ROWS_PER_TILE, ROWS_PER_TILE)])
    plsc.subcore_barrier()

    def body(i, carry):
        j2 = i + 2
        b2 = j2 % nbuf

        @pl.when(j2 < NCHUNK)
        def _():
            @pl.when(i >= nbuf - 2)
            def _():
                pltpu.make_async_copy(
                    rows.at[b2], acc.at[didxb.at[b2]], ssem.at[b2]).wait()
            fetch_idx(j2, b2)

        j1 = i + 1
        b1 = j1 % nbuf

        @pl.when(j1 < NCHUNK)
        def _():
            wait_idx(b1)
            pltpu.async_copy(xs_hbm.at[sidxb.at[b1]], rows.at[b1],
                             gsem.at[b1])

        b = i % nbuf
        pltpu.make_async_copy(xs_hbm.at[sidxb.at[b]], rows.at[b],
                              gsem.at[b]).wait()
        pltpu.async_copy(rows.at[b], acc.at[didxb.at[b]], ssem.at[b],
                         add=True)
        return carry

    lax.fori_loop(0, NCHUNK, body, 0)
    for b in range(nbuf):
        pltpu.make_async_copy(
            rows.at[b], acc.at[didxb.at[b]], ssem.at[b]).wait()
    plsc.subcore_barrier()
    pltpu.sync_copy(
        acc.at[pl.ds(s * ROWS_PER_TILE, ROWS_PER_TILE)],
        parts_out.at[c, pl.ds(s * ROWS_PER_TILE, ROWS_PER_TILE)],
    )


def _prop_body_staged(nbuf, xs_hbm, edges_hbm, zeros_hbm, parts_out,
                      acc, sidx2, didx2, rows, gsem, ssem):
    c = lax.axis_index("c")
    s = lax.axis_index("s")
    wid = c * NS + s
    pltpu.sync_copy(edges_hbm.at[0, wid], sidx2)
    pltpu.sync_copy(edges_hbm.at[1, wid], didx2)

    def gather(j, b):
        pltpu.async_copy(xs_hbm.at[sidx2.at[j]], rows.at[b], gsem.at[b])

    gather(0, 0)
    gather(1, 1)
    pltpu.sync_copy(zeros_hbm.at[0],
                    acc.at[pl.ds(s * ROWS_PER_TILE, ROWS_PER_TILE)])
    plsc.subcore_barrier()

    def body(i, carry):
        pre = i + 2
        bn = pre % nbuf

        @pl.when(pre < NCHUNK)
        def _():
            @pl.when(i >= nbuf - 2)
            def _():
                pltpu.make_async_copy(
                    rows.at[bn], acc.at[didx2.at[0]], ssem.at[bn]).wait()
            gather(pre, bn)

        b = i % nbuf
        pltpu.make_async_copy(xs_hbm.at[sidx2.at[0]], rows.at[b],
                              gsem.at[b]).wait()
        pltpu.async_copy(rows.at[b], acc.at[didx2.at[i]], ssem.at[b],
                         add=True)
        return carry

    lax.fori_loop(0, NCHUNK, body, 0)
    for b in range(nbuf):
        pltpu.make_async_copy(
            rows.at[b], acc.at[didx2.at[0]], ssem.at[b]).wait()
    plsc.subcore_barrier()
    pltpu.sync_copy(
        acc.at[pl.ds(s * ROWS_PER_TILE, ROWS_PER_TILE)],
        parts_out.at[c, pl.ds(s * ROWS_PER_TILE, ROWS_PER_TILE)],
    )


def _propagate(xs, edges, zeros2d, width):
    nbuf = 4
    if width == D_IN:
        body = functools.partial(_prop_body, nbuf)
        scratch = [
            pltpu.VMEM((nbuf, CHUNK), jnp.int32),
            pltpu.VMEM((nbuf, CHUNK), jnp.int32),
            pltpu.VMEM((nbuf, CHUNK, width), jnp.float32),
            pltpu.SemaphoreType.DMA((nbuf,)),
            pltpu.SemaphoreType.DMA((nbuf,)),
            pltpu.SemaphoreType.DMA((nbuf,)),
            pltpu.SemaphoreType.DMA((nbuf,)),
        ]
        use_tc_tiling = True
    else:
        nbuf = 6
        body = functools.partial(_prop_body_staged, nbuf)
        scratch = [
            pltpu.VMEM((NCHUNK, CHUNK), jnp.int32),
            pltpu.VMEM((NCHUNK, CHUNK), jnp.int32),
            pltpu.VMEM((nbuf, CHUNK, width), jnp.float32),
            pltpu.SemaphoreType.DMA((nbuf,)),
            pltpu.SemaphoreType.DMA((nbuf,)),
        ]
        use_tc_tiling = False
    k = pl.kernel(
        body,
        out_type=jax.ShapeDtypeStruct((NC, N_PAD, width), jnp.float32),
        mesh=plsc.VectorSubcoreMesh(core_axis_name="c", subcore_axis_name="s"),
        scratch_types=[
            pltpu.VMEM_SHARED((N_PAD, width), jnp.float32),
            *scratch,
        ],
        compiler_params=pltpu.CompilerParams(
            needs_layout_passes=False,
            use_tc_tiling_on_sc=use_tc_tiling),
    )
    return k(xs, edges, zeros2d)


def _norm_from_hist(h_blk):
    deg = jnp.sum(h_blk, axis=0)
    return lax.rsqrt(jnp.maximum(deg, 1.0))


def _scale_body(x_ref, hs_ref, o_ref):
    ns = _norm_from_hist(hs_ref[...])
    o_ref[...] = x_ref[...] * ns[:, None]


def _scale_x(x, hs):
    return pl.pallas_call(
        _scale_body,
        grid=(GRID_R,),
        in_specs=[
            pl.BlockSpec((R_BLK, D_IN), lambda i: (i, 0)),
            pl.BlockSpec((NW, R_BLK), lambda i: (0, i)),
        ],
        out_specs=pl.BlockSpec((R_BLK, D_IN), lambda i: (i, 0)),
        out_shape=jax.ShapeDtypeStruct((N, D_IN), jnp.float32),
    )(x, hs)


def _dense1_body(p0_ref, p1_ref, hd_ref, hs_ref, w1_ref, b1_ref, w2_ref,
                 h1_ref, y2_ref):
    nd = _norm_from_hist(hd_ref[...])
    ns = _norm_from_hist(hs_ref[...])
    agg = (p0_ref[0] + p1_ref[0]) * nd[:, None]
    h1 = jnp.dot(agg, w1_ref[...], preferred_element_type=jnp.float32)
    h1 = jnp.maximum(h1 + b1_ref[...], 0.0)
    h1_ref[...] = h1
    y2_ref[...] = jnp.dot(h1 * ns[:, None], w2_ref[...],
                          preferred_element_type=jnp.float32)


def _dense1(parts, hd, hs, w1, b1, w2p):
    return pl.pallas_call(
        _dense1_body,
        grid=(GRID_R,),
        in_specs=[
            pl.BlockSpec((1, R_BLK, D_H), lambda i: (0, i, 0)),
            pl.BlockSpec((1, R_BLK, D_H), lambda i: (1, i, 0)),
            pl.BlockSpec((NW, R_BLK), lambda i: (0, i)),
            pl.BlockSpec((NW, R_BLK), lambda i: (0, i)),
            pl.BlockSpec((D_IN, D_H), lambda i: (0, 0)),
            pl.BlockSpec((1, D_H), lambda i: (0, 0)),
            pl.BlockSpec((D_H, D_OUT_PAD), lambda i: (0, 0)),
        ],
        out_specs=[
            pl.BlockSpec((R_BLK, D_H), lambda i: (i, 0)),
            pl.BlockSpec((R_BLK, D_OUT_PAD), lambda i: (i, 0)),
        ],
        out_shape=[
            jax.ShapeDtypeStruct((N, D_H), jnp.float32),
            jax.ShapeDtypeStruct((N, D_OUT_PAD), jnp.float32),
        ],
    )(parts, parts, hd, hs, w1, b1, w2p)


def _dense2_body(q0_ref, q1_ref, hd_ref, b2_ref, out_ref, h2_ref):
    nd = _norm_from_hist(hd_ref[...])
    h2 = (q0_ref[0] + q1_ref[0]) * nd[:, None] + b2_ref[...]
    col = lax.broadcasted_iota(jnp.int32, h2.shape, 1)
    valid = col < D_OUT
    neg = jnp.full_like(h2, -jnp.inf)
    mx = jnp.max(jnp.where(valid, h2, neg), axis=1, keepdims=True)
    ex = jnp.where(valid, jnp.exp(h2 - mx), 0.0)
    lse = jnp.log(jnp.sum(ex, axis=1, keepdims=True)) + mx
    out = h2 - lse
    h2_ref[...] = h2[:, :D_OUT]
    out_ref[...] = out[:, :D_OUT]


def _dense2(parts, hd, b2p):
    return pl.pallas_call(
        _dense2_body,
        grid=(GRID_R,),
        in_specs=[
            pl.BlockSpec((1, R_BLK, D_OUT_PAD), lambda i: (0, i, 0)),
            pl.BlockSpec((1, R_BLK, D_OUT_PAD), lambda i: (1, i, 0)),
            pl.BlockSpec((NW, R_BLK), lambda i: (0, i)),
            pl.BlockSpec((1, D_OUT_PAD), lambda i: (0, 0)),
        ],
        out_specs=[
            pl.BlockSpec((R_BLK, D_OUT), lambda i: (i, 0)),
            pl.BlockSpec((R_BLK, D_OUT), lambda i: (i, 0)),
        ],
        out_shape=[
            jax.ShapeDtypeStruct((N, D_OUT), jnp.float32),
            jax.ShapeDtypeStruct((N, D_OUT), jnp.float32),
        ],
    )(parts, parts, hd, b2p)


@jax.jit
def kernel(x, graph, W1, b1, W2, b2):
    edges3 = graph.reshape(2, NW, NCHUNK, CHUNK)

    zeros1d = jnp.zeros((N_PAD,), jnp.float32)
    zeros128 = jnp.zeros((1, ROWS_PER_TILE, D_IN), jnp.float32)
    zeros48 = jnp.zeros((1, ROWS_PER_TILE, D_OUT_PAD), jnp.float32)

    hs, hd = _degrees(edges3, zeros1d)

    xs = _scale_x(x, hs)

    parts1 = _propagate(xs, edges3, zeros128, D_IN)

    w2p = jnp.pad(W2, ((0, 0), (0, D_OUT_PAD - D_OUT)))
    b1r = b1.reshape(1, D_H)
    h1, y2 = _dense1(parts1, hd, hs, W1, b1r, w2p)

    parts2 = _propagate(y2, edges3, zeros48, D_OUT_PAD)

    b2p = jnp.pad(b2, (0, D_OUT_PAD - D_OUT)).reshape(1, D_OUT_PAD)
    out, h2 = _dense2(parts2, hd, b2p)

    return (out, h1, h2)


# speedup vs baseline: 1.0153x; 1.0153x over previous
import functools

import jax
import jax.numpy as jnp
from jax import lax
from jax.experimental import pallas as pl
from jax.experimental.pallas import tpu as pltpu
from jax.experimental.pallas import tpu_sc as plsc

N = 10000
E = 320000
D_IN = 128
D_H = 128
D_OUT = 40
D_OUT_PAD = 48

NC = 2
NS = 16
NW = NC * NS
ROWS_PER_TILE = 640
N_PAD = NS * ROWS_PER_TILE
EDGES_PER_W = E // NW
CHUNK = 80
NCHUNK = EDGES_PER_W // CHUNK

R_BLK = 2048
GRID_R = 5


def _degree_body(edges_hbm, zeros_hbm, hs_out, hd_out, hist_s, hist_d,
                 sidx2, didx2):
    c = lax.axis_index("c")
    s = lax.axis_index("s")
    wid = c * NS + s
    pltpu.sync_copy(zeros_hbm, hist_s)
    pltpu.sync_copy(zeros_hbm, hist_d)
    pltpu.sync_copy(edges_hbm.at[0, wid], sidx2)
    pltpu.sync_copy(edges_hbm.at[1, wid], didx2)
    ones = jnp.ones((16,), jnp.float32)

    def body(i, carry):
        for j in range(CHUNK // 16):
            si = sidx2[i, pl.ds(j * 16, 16)]
            di = didx2[i, pl.ds(j * 16, 16)]
            plsc.addupdate_scatter(hist_s, [si], ones)
            plsc.addupdate_scatter(hist_d, [di], ones)
        return carry

    lax.fori_loop(0, NCHUNK, body, 0)
    pltpu.sync_copy(hist_s, hs_out.at[wid])
    pltpu.sync_copy(hist_d, hd_out.at[wid])


def _degrees(edges3, zeros1d):
    k = pl.kernel(
        _degree_body,
        out_type=(
            jax.ShapeDtypeStruct((NW, N_PAD), jnp.float32),
            jax.ShapeDtypeStruct((NW, N_PAD), jnp.float32),
        ),
        mesh=plsc.VectorSubcoreMesh(core_axis_name="c", subcore_axis_name="s"),
        scratch_types=[
            pltpu.VMEM((N_PAD,), jnp.float32),
            pltpu.VMEM((N_PAD,), jnp.float32),
            pltpu.VMEM((NCHUNK, CHUNK), jnp.int32),
            pltpu.VMEM((NCHUNK, CHUNK), jnp.int32),
        ],
        compiler_params=pltpu.CompilerParams(needs_layout_passes=False),
    )
    return k(edges3, zeros1d)


def _prop_body(nbuf, xs_hbm, edges_hbm, zeros_hbm, parts_out,
               acc, sidxb, didxb, rows, isems, isemd, gsem, ssem):
    c = lax.axis_index("c")
    s = lax.axis_index("s")
    wid = c * NS + s
    def fetch_idx(j, b):
        pltpu.async_copy(edges_hbm.at[0, wid, j], sidxb.at[b], isems.at[b])
        pltpu.async_copy(edges_hbm.at[1, wid, j], didxb.at[b], isemd.at[b])

    def wait_idx(b):
        pltpu.make_async_copy(edges_hbm.at[0, wid, 0], sidxb.at[b],
                              isems.at[b]).wait()
        pltpu.make_async_copy(edges_hbm.at[1, wid, 0], didxb.at[b],
                              isemd.at[b]).wait()

    fetch_idx(0, 0)
    fetch_idx(1, 1)
    wait_idx(0)
    pltpu.async_copy(xs_hbm.at[sidxb.at[0]], rows.at[0], gsem.at[0])
    pltpu.sync_copy(zeros_hbm.at[0],
                    acc.at[pl.ds(s * ROWS_PER_TILE, ROWS_PER_TILE)])
    plsc.subcore_barrier()

    def body(i, carry):
        j2 = i + 2
        b2 = j2 % nbuf

        @pl.when(j2 < NCHUNK)
        def _():
            @pl.when(i >= nbuf - 2)
            def _():
                pltpu.make_async_copy(
                    rows.at[b2], acc.at[didxb.at[b2]], ssem.at[b2]).wait()
            fetch_idx(j2, b2)

        j1 = i + 1
        b1 = j1 % nbuf

        @pl.when(j1 < NCHUNK)
        def _():
            wait_idx(b1)
            pltpu.async_copy(xs_hbm.at[sidxb.at[b1]], rows.at[b1],
                             gsem.at[b1])

        b = i % nbuf
        pltpu.make_async_copy(xs_hbm.at[sidxb.at[b]], rows.at[b],
                              gsem.at[b]).wait()
        pltpu.async_copy(rows.at[b], acc.at[didxb.at[b]], ssem.at[b],
                         add=True)
        return carry

    lax.fori_loop(0, NCHUNK, body, 0)
    for b in range(nbuf):
        pltpu.make_async_copy(
            rows.at[b], acc.at[didxb.at[b]], ssem.at[b]).wait()
    plsc.subcore_barrier()
    pltpu.sync_copy(
        acc.at[pl.ds(s * ROWS_PER_TILE, ROWS_PER_TILE)],
        parts_out.at[c, pl.ds(s * ROWS_PER_TILE, ROWS_PER_TILE)],
    )


def _prop_body_staged(nbuf, xs_hbm, edges_hbm, zeros_hbm, parts_out,
                      acc, sidx2, didx2, rows, gsem, ssem):
    c = lax.axis_index("c")
    s = lax.axis_index("s")
    wid = c * NS + s
    pltpu.sync_copy(edges_hbm.at[0, wid], sidx2)
    pltpu.sync_copy(edges_hbm.at[1, wid], didx2)

    def gather(j, b):
        pltpu.async_copy(xs_hbm.at[sidx2.at[j]], rows.at[b], gsem.at[b])

    gather(0, 0)
    gather(1, 1)
    pltpu.sync_copy(zeros_hbm.at[0],
                    acc.at[pl.ds(s * ROWS_PER_TILE, ROWS_PER_TILE)])
    plsc.subcore_barrier()

    def body(i, carry):
        pre = i + 2
        bn = pre % nbuf

        @pl.when(pre < NCHUNK)
        def _():
            @pl.when(i >= nbuf - 2)
            def _():
                pltpu.make_async_copy(
                    rows.at[bn], acc.at[didx2.at[0]], ssem.at[bn]).wait()
            gather(pre, bn)

        b = i % nbuf
        pltpu.make_async_copy(xs_hbm.at[sidx2.at[0]], rows.at[b],
                              gsem.at[b]).wait()
        pltpu.async_copy(rows.at[b], acc.at[didx2.at[i]], ssem.at[b],
                         add=True)
        return carry

    lax.fori_loop(0, NCHUNK, body, 0)
    for b in range(nbuf):
        pltpu.make_async_copy(
            rows.at[b], acc.at[didx2.at[0]], ssem.at[b]).wait()
    plsc.subcore_barrier()
    pltpu.sync_copy(
        acc.at[pl.ds(s * ROWS_PER_TILE, ROWS_PER_TILE)],
        parts_out.at[c, pl.ds(s * ROWS_PER_TILE, ROWS_PER_TILE)],
    )


def _propagate(xs, edges, zeros2d, width):
    nbuf = 4
    if width == D_IN:
        body = functools.partial(_prop_body, nbuf)
        scratch = [
            pltpu.VMEM((nbuf, CHUNK), jnp.int32),
            pltpu.VMEM((nbuf, CHUNK), jnp.int32),
            pltpu.VMEM((nbuf, CHUNK, width), jnp.float32),
            pltpu.SemaphoreType.DMA((nbuf,)),
            pltpu.SemaphoreType.DMA((nbuf,)),
            pltpu.SemaphoreType.DMA((nbuf,)),
            pltpu.SemaphoreType.DMA((nbuf,)),
        ]
        use_tc_tiling = True
    else:
        body = functools.partial(_prop_body_staged, nbuf)
        scratch = [
            pltpu.VMEM((NCHUNK, CHUNK), jnp.int32),
            pltpu.VMEM((NCHUNK, CHUNK), jnp.int32),
            pltpu.VMEM((nbuf, CHUNK, width), jnp.float32),
            pltpu.SemaphoreType.DMA((nbuf,)),
            pltpu.SemaphoreType.DMA((nbuf,)),
        ]
        use_tc_tiling = False
    k = pl.kernel(
        body,
        out_type=jax.ShapeDtypeStruct((NC, N_PAD, width), jnp.float32),
        mesh=plsc.VectorSubcoreMesh(core_axis_name="c", subcore_axis_name="s"),
        scratch_types=[
            pltpu.VMEM_SHARED((N_PAD, width), jnp.float32),
            *scratch,
        ],
        compiler_params=pltpu.CompilerParams(
            needs_layout_passes=False,
            use_tc_tiling_on_sc=use_tc_tiling),
    )
    return k(xs, edges, zeros2d)


def _norm_from_hist(h_blk):
    deg = jnp.sum(h_blk, axis=0)
    return lax.rsqrt(jnp.maximum(deg, 1.0))


def _scale_body(x_ref, hs_ref, o_ref):
    ns = _norm_from_hist(hs_ref[...])
    o_ref[...] = x_ref[...] * ns[:, None]


def _scale_x(x, hs):
    return pl.pallas_call(
        _scale_body,
        grid=(GRID_R,),
        in_specs=[
            pl.BlockSpec((R_BLK, D_IN), lambda i: (i, 0)),
            pl.BlockSpec((NW, R_BLK), lambda i: (0, i)),
        ],
        out_specs=pl.BlockSpec((R_BLK, D_IN), lambda i: (i, 0)),
        out_shape=jax.ShapeDtypeStruct((N, D_IN), jnp.float32),
    )(x, hs)


def _dense1_body(p0_ref, p1_ref, hd_ref, hs_ref, w1_ref, b1_ref, w2_ref,
                 h1_ref, y2_ref):
    nd = _norm_from_hist(hd_ref[...])
    ns = _norm_from_hist(hs_ref[...])
    agg = (p0_ref[0] + p1_ref[0]) * nd[:, None]
    h1 = jnp.dot(agg, w1_ref[...], preferred_element_type=jnp.float32)
    h1 = jnp.maximum(h1 + b1_ref[...], 0.0)
    h1_ref[...] = h1
    y2_ref[...] = jnp.dot(h1 * ns[:, None], w2_ref[...],
                          preferred_element_type=jnp.float32)


def _dense1(parts, hd, hs, w1, b1, w2p):
    return pl.pallas_call(
        _dense1_body,
        grid=(GRID_R,),
        in_specs=[
            pl.BlockSpec((1, R_BLK, D_H), lambda i: (0, i, 0)),
            pl.BlockSpec((1, R_BLK, D_H), lambda i: (1, i, 0)),
            pl.BlockSpec((NW, R_BLK), lambda i: (0, i)),
            pl.BlockSpec((NW, R_BLK), lambda i: (0, i)),
            pl.BlockSpec((D_IN, D_H), lambda i: (0, 0)),
            pl.BlockSpec((1, D_H), lambda i: (0, 0)),
            pl.BlockSpec((D_H, D_OUT_PAD), lambda i: (0, 0)),
        ],
        out_specs=[
            pl.BlockSpec((R_BLK, D_H), lambda i: (i, 0)),
            pl.BlockSpec((R_BLK, D_OUT_PAD), lambda i: (i, 0)),
        ],
        out_shape=[
            jax.ShapeDtypeStruct((N, D_H), jnp.float32),
            jax.ShapeDtypeStruct((N, D_OUT_PAD), jnp.float32),
        ],
    )(parts, parts, hd, hs, w1, b1, w2p)


def _dense2_body(q0_ref, q1_ref, hd_ref, b2_ref, out_ref, h2_ref):
    nd = _norm_from_hist(hd_ref[...])
    h2 = (q0_ref[0] + q1_ref[0]) * nd[:, None] + b2_ref[...]
    col = lax.broadcasted_iota(jnp.int32, h2.shape, 1)
    valid = col < D_OUT
    neg = jnp.full_like(h2, -jnp.inf)
    mx = jnp.max(jnp.where(valid, h2, neg), axis=1, keepdims=True)
    ex = jnp.where(valid, jnp.exp(h2 - mx), 0.0)
    lse = jnp.log(jnp.sum(ex, axis=1, keepdims=True)) + mx
    out = h2 - lse
    h2_ref[...] = h2[:, :D_OUT]
    out_ref[...] = out[:, :D_OUT]


def _dense2(parts, hd, b2p):
    return pl.pallas_call(
        _dense2_body,
        grid=(GRID_R,),
        in_specs=[
            pl.BlockSpec((1, R_BLK, D_OUT_PAD), lambda i: (0, i, 0)),
            pl.BlockSpec((1, R_BLK, D_OUT_PAD), lambda i: (1, i, 0)),
            pl.BlockSpec((NW, R_BLK), lambda i: (0, i)),
            pl.BlockSpec((1, D_OUT_PAD), lambda i: (0, 0)),
        ],
        out_specs=[
            pl.BlockSpec((R_BLK, D_OUT), lambda i: (i, 0)),
            pl.BlockSpec((R_BLK, D_OUT), lambda i: (i, 0)),
        ],
        out_shape=[
            jax.ShapeDtypeStruct((N, D_OUT), jnp.float32),
            jax.ShapeDtypeStruct((N, D_OUT), jnp.float32),
        ],
    )(parts, parts, hd, b2p)


@jax.jit
def kernel(x, graph, W1, b1, W2, b2):
    edges3 = graph.reshape(2, NW, NCHUNK, CHUNK)

    zeros1d = jnp.zeros((N_PAD,), jnp.float32)
    zeros128 = jnp.zeros((1, ROWS_PER_TILE, D_IN), jnp.float32)
    zeros48 = jnp.zeros((1, ROWS_PER_TILE, D_OUT_PAD), jnp.float32)

    hs, hd = _degrees(edges3, zeros1d)

    xs = _scale_x(x, hs)

    parts1 = _propagate(xs, edges3, zeros128, D_IN)

    w2p = jnp.pad(W2, ((0, 0), (0, D_OUT_PAD - D_OUT)))
    b1r = b1.reshape(1, D_H)
    h1, y2 = _dense1(parts1, hd, hs, W1, b1r, w2p)

    parts2 = _propagate(y2, edges3, zeros48, D_OUT_PAD)

    b2p = jnp.pad(b2, (0, D_OUT_PAD - D_OUT)).reshape(1, D_OUT_PAD)
    out, h2 = _dense2(parts2, hd, b2p)

    return (out, h1, h2)
